# baseline (speedup 1.0000x reference)
import numpy as np

import jax
import jax.numpy as jnp
from jax import lax
from jax.experimental import pallas as pl
from jax.experimental.pallas import tpu as pltpu
from jax.experimental.pallas import tpu_sc as plsc

NUM_CLASSES = 80
MAX_BOXES = 20
SCORE_THR = 0.6
IOU_THR = 0.5

_G0, _G1 = 19, 38
_N0 = _G0 * _G0 * 3
_N1 = _G1 * _G1 * 3
_N = _N0 + _N1
_NP = 43 * 128
_NCHUNK = _NP // 16
_NPC = _NP + 32

_ANCHORS = np.array([[10.0, 14.0], [23.0, 27.0], [37.0, 58.0],
                     [81.0, 82.0], [135.0, 169.0], [344.0, 319.0]],
                    dtype=np.float32)
_MASK0 = [3, 4, 5]
_MASK1 = [1, 2, 3]

_INPUT = np.float32(_G0 * 32)
_IMG = np.array([720.0, 1280.0], dtype=np.float32)
_SC_ = np.float32(min(np.float32(_INPUT / _IMG[0]), np.float32(_INPUT / _IMG[1])))
_NEW = np.round(_IMG * _SC_).astype(np.float32)
_OFF = ((_INPUT - _NEW) / np.float32(2.0) / _INPUT).astype(np.float32)
_SCALE = (_INPUT / _NEW).astype(np.float32)


def _make_consts():
    gx = np.zeros(_NP, np.float32)
    gy = np.zeros(_NP, np.float32)
    gd = np.ones(_NP, np.float32)
    aw = np.zeros(_NP, np.float32)
    ah = np.zeros(_NP, np.float32)
    n = np.arange(_N0)
    cell, a = n // 3, n % 3
    gx[:_N0] = (cell % _G0).astype(np.float32)
    gy[:_N0] = (cell // _G0).astype(np.float32)
    gd[:_N0] = float(_G0)
    anc = _ANCHORS[_MASK0][a]
    aw[:_N0] = anc[:, 0]
    ah[:_N0] = anc[:, 1]
    n = np.arange(_N1)
    cell, a = n // 3, n % 3
    gx[_N0:_N] = (cell % _G1).astype(np.float32)
    gy[_N0:_N] = (cell // _G1).astype(np.float32)
    gd[_N0:_N] = float(_G1)
    anc = _ANCHORS[_MASK1][a]
    aw[_N0:_N] = anc[:, 0]
    ah[_N0:_N] = anc[:, 1]
    return np.stack([gx, gy, gd, aw, ah], axis=0)


_CONSTS = _make_consts()


def _decode_body(t_ref, c_ref, s_out, b_out):
    def sig(x):
        return 1.0 / (1.0 + jnp.exp(-x))

    tx = t_ref[0:1, :]
    ty = t_ref[1:2, :]
    tw = t_ref[2:3, :]
    th = t_ref[3:4, :]
    tc = t_ref[4:5, :]
    tp = t_ref[5:85, :]
    gx = c_ref[0:1, :]
    gy = c_ref[1:2, :]
    gd = c_ref[2:3, :]
    aw = c_ref[3:4, :]
    ah = c_ref[4:5, :]

    bx = (sig(tx) + gx) / gd
    by = (sig(ty) + gy) / gd
    bw = jnp.exp(tw) * aw / _INPUT
    bh = jnp.exp(th) * ah / _INPUT
    yy = (by - _OFF[0]) * _SCALE[0]
    xx = (bx - _OFF[1]) * _SCALE[1]
    hh = bh * _SCALE[0]
    ww = bw * _SCALE[1]
    y1 = (yy - hh / 2.0) * _IMG[0]
    x1 = (xx - ww / 2.0) * _IMG[1]
    y2 = (yy + hh / 2.0) * _IMG[0]
    x2 = (xx + ww / 2.0) * _IMG[1]
    area = (y2 - y1) * (x2 - x1)

    s = sig(tc) * sig(tp)
    s = jnp.where(s >= SCORE_THR, s, 0.0)

    s_out[:, :] = s
    b_out[0:1, :] = y1
    b_out[1:2, :] = x1
    b_out[2:3, :] = y2
    b_out[3:4, :] = x2
    b_out[4:5, :] = area


def _sc_nms_body(s_hbm, b_hbm, out_hbm,
                 s_v, y1_v, x1_v, y2_v, x2_v, ar_v, cs_v, ci_v, sel_v,
                 shf_v, shi_v):
    wid = lax.axis_index("s") * 2 + lax.axis_index("c")
    lane = jnp.arange(16, dtype=jnp.int32)

    def bmax(x):
        for s in (8, 4, 2, 1):
            shf_v[pl.ds(0, 16)] = x
            x = jnp.maximum(x, plsc.load_gather(shf_v, [lane ^ s]))
        return x

    def bmin(x):
        for s in (8, 4, 2, 1):
            shi_v[pl.ds(0, 16)] = x
            x = jnp.minimum(x, plsc.load_gather(shi_v, [lane ^ s]))
        return x
    zero16 = jnp.zeros((16,), jnp.float32)
    izero16 = jnp.zeros((16,), jnp.int32)
    bv0 = jnp.full((16,), -1.0, jnp.float32)
    bi0 = jnp.zeros((16,), jnp.int32)

    pltpu.sync_copy(b_hbm.at[pl.ds(0 * _NP, _NP)], y1_v)
    pltpu.sync_copy(b_hbm.at[pl.ds(1 * _NP, _NP)], x1_v)
    pltpu.sync_copy(b_hbm.at[pl.ds(2 * _NP, _NP)], y2_v)
    pltpu.sync_copy(b_hbm.at[pl.ds(3 * _NP, _NP)], x2_v)
    pltpu.sync_copy(b_hbm.at[pl.ds(4 * _NP, _NP)], ar_v)

    def do_class(c):
        pltpu.sync_copy(s_hbm.at[pl.ds(c * _NP, _NP)], s_v)
        for k in range(10):
            sel_v[pl.ds(k * 16, 16)] = zero16

        @pl.loop(0, _NCHUNK, init_carry=jnp.int32(0), unroll=1)
        def comp(k, cnt):
            v = s_v[pl.ds(k * 16, 16)]
            msk = v > 0.0
            n = plsc.all_reduce_population_count(msk)
            nk = n[0]

            @pl.when(nk > 0)
            def _():
                mi = msk.astype(jnp.int32)
                cum = mi
                for s in (1, 2, 4, 8):
                    shi_v[pl.ds(0, 16)] = cum
                    sh = plsc.load_gather(shi_v, [jnp.maximum(lane - s, 0)])
                    cum = cum + jnp.where(lane >= s, sh, 0)
                pos = jnp.where(msk, cnt + cum - 1, jnp.int32(_NP) + lane)
                plsc.store_scatter(cs_v, [pos], v)
                plsc.store_scatter(ci_v, [pos], k * 16 + lane)

            return cnt + nk

        cnt = comp
        plsc.store_scatter(cs_v, [cnt + lane], zero16)
        plsc.store_scatter(cs_v, [cnt + 16 + lane], zero16)
        plsc.store_scatter(ci_v, [cnt + lane], izero16)
        plsc.store_scatter(ci_v, [cnt + 16 + lane], izero16)
        nb = (cnt + 15) // 16

        @pl.loop(0, nb, init_carry=(bv0, bi0), unroll=1)
        def amx(k, carry):
            bv, bi = carry
            v = cs_v[pl.ds(k * 16, 16)]
            gt = v > bv
            return jnp.where(gt, v, bv), jnp.where(gt, k, bi)

        bv, bi = amx

        @pl.loop(0, MAX_BOXES, init_carry=(bv, bi), unroll=1)
        def rnd(i, carry):
            bv, bi = carry
            mx_v = bmax(bv)
            idxv = bi * 16 + lane
            cand_v = bmin(jnp.where(bv == mx_v, idxv, jnp.int32(1 << 30)))
            keep_v = mx_v > 0.0
            oi_v = plsc.load_gather(ci_v, [cand_v])
            ky1 = plsc.load_gather(y1_v, [oi_v])
            kx1 = plsc.load_gather(x1_v, [oi_v])
            ky2 = plsc.load_gather(y2_v, [oi_v])
            kx2 = plsc.load_gather(x2_v, [oi_v])
            kar = plsc.load_gather(ar_v, [oi_v])
            val = jnp.where(lane == 0, mx_v,
                            jnp.where(lane == 1, ky1,
                                      jnp.where(lane == 2, kx1,
                                                jnp.where(lane == 3, ky2, kx2))))
            selpos = jnp.where((lane < 5) & keep_v, lane * 32 + i,
                               jnp.int32(176) + lane)
            plsc.store_scatter(sel_v, [selpos], val)

            @pl.loop(0, nb, init_carry=(bv0, bi0), unroll=1)
            def sup(k, carry2):
                nv, ni = carry2
                sl = pl.ds(k * 16, 16)
                sv = cs_v[sl]
                gi = ci_v[sl]
                vy1 = plsc.load_gather(y1_v, [gi])
                vx1 = plsc.load_gather(x1_v, [gi])
                vy2 = plsc.load_gather(y2_v, [gi])
                vx2 = plsc.load_gather(x2_v, [gi])
                var_ = plsc.load_gather(ar_v, [gi])
                iy1 = jnp.maximum(ky1, vy1)
                ix1 = jnp.maximum(kx1, vx1)
                iy2 = jnp.minimum(ky2, vy2)
                ix2 = jnp.minimum(kx2, vx2)
                inter = jnp.maximum(iy2 - iy1, 0.0) * jnp.maximum(ix2 - ix1, 0.0)
                iou = inter / (kar + var_ - inter + 1e-9)
                pos = k * 16 + lane
                supm = (iou > IOU_THR) | (pos == cand_v)
                s_upd = jnp.where(supm & keep_v, 0.0, sv)
                cs_v[sl] = s_upd
                gt = s_upd > nv
                return jnp.where(gt, s_upd, nv), jnp.where(gt, k, ni)

            return sup

        pltpu.sync_copy(sel_v.at[pl.ds(0, 160)], out_hbm.at[pl.ds(c * 160, 160)])

    @pl.loop(0, 3, unroll=1)
    def rep_body(rep):
        c = wid + rep * 32

        @pl.when(c < NUM_CLASSES)
        def _():
            do_class(c)


def kernel(feat0, feat1, image_shape_t):
    del image_shape_t
    f0 = feat0.reshape(_N0, NUM_CLASSES + 5)
    f1 = feat1.reshape(_N1, NUM_CLASSES + 5)
    t = jnp.concatenate([f0, f1], axis=0)
    t = jnp.pad(t, ((0, _NP - _N), (0, 0)), constant_values=-1e9).T
    consts = jnp.asarray(_CONSTS)

    s_mat, b_mat = pl.pallas_call(
        _decode_body,
        out_shape=[jax.ShapeDtypeStruct((NUM_CLASSES, _NP), jnp.float32),
                   jax.ShapeDtypeStruct((5, _NP), jnp.float32)],
    )(t, consts)

    mesh = plsc.VectorSubcoreMesh(core_axis_name="c", subcore_axis_name="s",
                                  num_cores=2, num_subcores=16)
    out = pl.kernel(
        _sc_nms_body,
        out_type=jax.ShapeDtypeStruct((NUM_CLASSES * 160,), jnp.float32),
        mesh=mesh,
        compiler_params=pltpu.CompilerParams(needs_layout_passes=False),
        scratch_types=[
            pltpu.VMEM((_NP,), jnp.float32),
            pltpu.VMEM((_NP,), jnp.float32),
            pltpu.VMEM((_NP,), jnp.float32),
            pltpu.VMEM((_NP,), jnp.float32),
            pltpu.VMEM((_NP,), jnp.float32),
            pltpu.VMEM((_NP,), jnp.float32),
            pltpu.VMEM((_NPC,), jnp.float32),
            pltpu.VMEM((_NPC,), jnp.int32),
            pltpu.VMEM((192,), jnp.float32),
            pltpu.VMEM((16,), jnp.float32),
            pltpu.VMEM((16,), jnp.int32),
        ],
    )(s_mat.reshape(-1), b_mat.reshape(-1))

    sel = out.reshape(NUM_CLASSES, 5, 32)[:, :, :MAX_BOXES]
    boxes_ = jnp.stack([sel[:, 1], sel[:, 2], sel[:, 3], sel[:, 4]],
                       axis=-1).reshape(-1, 4)
    scores_ = sel[:, 0].reshape(-1)
    classes_ = jnp.repeat(jnp.arange(NUM_CLASSES, dtype=jnp.int32), MAX_BOXES)
    return boxes_, scores_, classes_


# speedup vs baseline: 1.1587x; 1.1587x over previous
import numpy as np

import jax
import jax.numpy as jnp
from jax import lax
from jax.experimental import pallas as pl
from jax.experimental.pallas import tpu as pltpu
from jax.experimental.pallas import tpu_sc as plsc

NUM_CLASSES = 80
MAX_BOXES = 20
SCORE_THR = 0.6
IOU_THR = 0.5

_G0, _G1 = 19, 38
_N0 = _G0 * _G0 * 3
_N1 = _G1 * _G1 * 3
_N = _N0 + _N1
_NP = 43 * 128
_NCHUNK = _NP // 16
_NPC = _NP + 32

_ANCHORS = np.array([[10.0, 14.0], [23.0, 27.0], [37.0, 58.0],
                     [81.0, 82.0], [135.0, 169.0], [344.0, 319.0]],
                    dtype=np.float32)
_MASK0 = [3, 4, 5]
_MASK1 = [1, 2, 3]

_INPUT = np.float32(_G0 * 32)
_IMG = np.array([720.0, 1280.0], dtype=np.float32)
_SC_ = np.float32(min(np.float32(_INPUT / _IMG[0]), np.float32(_INPUT / _IMG[1])))
_NEW = np.round(_IMG * _SC_).astype(np.float32)
_OFF = ((_INPUT - _NEW) / np.float32(2.0) / _INPUT).astype(np.float32)
_SCALE = (_INPUT / _NEW).astype(np.float32)


def _make_consts():
    gx = np.zeros(_NP, np.float32)
    gy = np.zeros(_NP, np.float32)
    gd = np.ones(_NP, np.float32)
    aw = np.zeros(_NP, np.float32)
    ah = np.zeros(_NP, np.float32)
    n = np.arange(_N0)
    cell, a = n // 3, n % 3
    gx[:_N0] = (cell % _G0).astype(np.float32)
    gy[:_N0] = (cell // _G0).astype(np.float32)
    gd[:_N0] = float(_G0)
    anc = _ANCHORS[_MASK0][a]
    aw[:_N0] = anc[:, 0]
    ah[:_N0] = anc[:, 1]
    n = np.arange(_N1)
    cell, a = n // 3, n % 3
    gx[_N0:_N] = (cell % _G1).astype(np.float32)
    gy[_N0:_N] = (cell // _G1).astype(np.float32)
    gd[_N0:_N] = float(_G1)
    anc = _ANCHORS[_MASK1][a]
    aw[_N0:_N] = anc[:, 0]
    ah[_N0:_N] = anc[:, 1]
    return np.stack([gx, gy, gd, aw, ah], axis=0)


_CONSTS = _make_consts()


def _decode_body(t_ref, c_ref, s_out, b_out):
    def sig(x):
        return 1.0 / (1.0 + jnp.exp(-x))

    tx = t_ref[0:1, :]
    ty = t_ref[1:2, :]
    tw = t_ref[2:3, :]
    th = t_ref[3:4, :]
    tc = t_ref[4:5, :]
    tp = t_ref[5:85, :]
    gx = c_ref[0:1, :]
    gy = c_ref[1:2, :]
    gd = c_ref[2:3, :]
    aw = c_ref[3:4, :]
    ah = c_ref[4:5, :]

    bx = (sig(tx) + gx) / gd
    by = (sig(ty) + gy) / gd
    bw = jnp.exp(tw) * aw / _INPUT
    bh = jnp.exp(th) * ah / _INPUT
    yy = (by - _OFF[0]) * _SCALE[0]
    xx = (bx - _OFF[1]) * _SCALE[1]
    hh = bh * _SCALE[0]
    ww = bw * _SCALE[1]
    y1 = (yy - hh / 2.0) * _IMG[0]
    x1 = (xx - ww / 2.0) * _IMG[1]
    y2 = (yy + hh / 2.0) * _IMG[0]
    x2 = (xx + ww / 2.0) * _IMG[1]
    area = (y2 - y1) * (x2 - x1)

    s = sig(tc) * sig(tp)
    s = jnp.where(s >= SCORE_THR, s, 0.0)

    s_out[:, :] = s
    b_out[0:1, :] = y1
    b_out[1:2, :] = x1
    b_out[2:3, :] = y2
    b_out[3:4, :] = x2
    b_out[4:5, :] = area


def _sc_nms_body(s_hbm, b_hbm, out_hbm,
                 s_v, y1_v, x1_v, y2_v, x2_v, ar_v, cs_v, ci_v, sel_v,
                 shf_v, shi_v):
    wid = lax.axis_index("s") * 2 + lax.axis_index("c")
    lane = jnp.arange(16, dtype=jnp.int32)

    def bmax(x):
        for s in (8, 4, 2, 1):
            shf_v[pl.ds(0, 16)] = x
            x = jnp.maximum(x, plsc.load_gather(shf_v, [lane ^ s]))
        return x

    def bmin(x):
        for s in (8, 4, 2, 1):
            shi_v[pl.ds(0, 16)] = x
            x = jnp.minimum(x, plsc.load_gather(shi_v, [lane ^ s]))
        return x
    zero16 = jnp.zeros((16,), jnp.float32)
    izero16 = jnp.zeros((16,), jnp.int32)
    bv0 = jnp.full((16,), -1.0, jnp.float32)
    bi0 = jnp.zeros((16,), jnp.int32)

    pltpu.sync_copy(b_hbm.at[pl.ds(0 * _NP, _NP)], y1_v)
    pltpu.sync_copy(b_hbm.at[pl.ds(1 * _NP, _NP)], x1_v)
    pltpu.sync_copy(b_hbm.at[pl.ds(2 * _NP, _NP)], y2_v)
    pltpu.sync_copy(b_hbm.at[pl.ds(3 * _NP, _NP)], x2_v)
    pltpu.sync_copy(b_hbm.at[pl.ds(4 * _NP, _NP)], ar_v)

    def do_class(c):
        pltpu.sync_copy(s_hbm.at[pl.ds(c * _NP, _NP)], s_v)
        for k in range(10):
            sel_v[pl.ds(k * 16, 16)] = zero16

        @pl.loop(0, _NCHUNK, init_carry=jnp.int32(0), unroll=4)
        def comp(k, cnt):
            v = s_v[pl.ds(k * 16, 16)]
            msk = v > 0.0
            mi = msk.astype(jnp.int32)
            cum = mi
            for s in (1, 2, 4, 8):
                shi_v[pl.ds(0, 16)] = cum
                sh = plsc.load_gather(shi_v, [jnp.maximum(lane - s, 0)])
                cum = cum + jnp.where(lane >= s, sh, 0)
            pos = jnp.where(msk, cnt + cum - 1, jnp.int32(_NP) + lane)
            plsc.store_scatter(cs_v, [pos], v)
            plsc.store_scatter(ci_v, [pos], k * 16 + lane)
            n = plsc.all_reduce_population_count(msk)
            return cnt + n[0]

        cnt = comp
        plsc.store_scatter(cs_v, [cnt + lane], zero16)
        plsc.store_scatter(cs_v, [cnt + 16 + lane], zero16)
        plsc.store_scatter(ci_v, [cnt + lane], izero16)
        plsc.store_scatter(ci_v, [cnt + 16 + lane], izero16)
        nb = (cnt + 15) // 16

        @pl.loop(0, nb, init_carry=(bv0, bi0), unroll=1)
        def amx(k, carry):
            bv, bi = carry
            v = cs_v[pl.ds(k * 16, 16)]
            gt = v > bv
            return jnp.where(gt, v, bv), jnp.where(gt, k, bi)

        bv, bi = amx

        @pl.loop(0, MAX_BOXES, init_carry=(bv, bi), unroll=1)
        def rnd(i, carry):
            bv, bi = carry
            mx_v = bmax(bv)
            idxv = bi * 16 + lane
            cand_v = bmin(jnp.where(bv == mx_v, idxv, jnp.int32(1 << 30)))
            keep_v = mx_v > 0.0
            oi_v = plsc.load_gather(ci_v, [cand_v])
            ky1 = plsc.load_gather(y1_v, [oi_v])
            kx1 = plsc.load_gather(x1_v, [oi_v])
            ky2 = plsc.load_gather(y2_v, [oi_v])
            kx2 = plsc.load_gather(x2_v, [oi_v])
            kar = plsc.load_gather(ar_v, [oi_v])
            val = jnp.where(lane == 0, mx_v,
                            jnp.where(lane == 1, ky1,
                                      jnp.where(lane == 2, kx1,
                                                jnp.where(lane == 3, ky2, kx2))))
            selpos = jnp.where((lane < 5) & keep_v, lane * 32 + i,
                               jnp.int32(176) + lane)
            plsc.store_scatter(sel_v, [selpos], val)

            @pl.loop(0, nb, init_carry=(bv0, bi0), unroll=1)
            def sup(k, carry2):
                nv, ni = carry2
                sl = pl.ds(k * 16, 16)
                sv = cs_v[sl]
                gi = ci_v[sl]
                vy1 = plsc.load_gather(y1_v, [gi])
                vx1 = plsc.load_gather(x1_v, [gi])
                vy2 = plsc.load_gather(y2_v, [gi])
                vx2 = plsc.load_gather(x2_v, [gi])
                var_ = plsc.load_gather(ar_v, [gi])
                iy1 = jnp.maximum(ky1, vy1)
                ix1 = jnp.maximum(kx1, vx1)
                iy2 = jnp.minimum(ky2, vy2)
                ix2 = jnp.minimum(kx2, vx2)
                inter = jnp.maximum(iy2 - iy1, 0.0) * jnp.maximum(ix2 - ix1, 0.0)
                iou = inter / (kar + var_ - inter + 1e-9)
                pos = k * 16 + lane
                supm = (iou > IOU_THR) | (pos == cand_v)
                s_upd = jnp.where(supm & keep_v, 0.0, sv)
                cs_v[sl] = s_upd
                gt = s_upd > nv
                return jnp.where(gt, s_upd, nv), jnp.where(gt, k, ni)

            return sup

        pltpu.sync_copy(sel_v.at[pl.ds(0, 160)], out_hbm.at[pl.ds(c * 160, 160)])

    @pl.loop(0, 3, unroll=1)
    def rep_body(rep):
        c = wid + rep * 32

        @pl.when(c < NUM_CLASSES)
        def _():
            do_class(c)


def kernel(feat0, feat1, image_shape_t):
    del image_shape_t
    f0 = feat0.reshape(_N0, NUM_CLASSES + 5)
    f1 = feat1.reshape(_N1, NUM_CLASSES + 5)
    t = jnp.concatenate([f0, f1], axis=0)
    t = jnp.pad(t, ((0, _NP - _N), (0, 0)), constant_values=-1e9).T
    consts = jnp.asarray(_CONSTS)

    s_mat, b_mat = pl.pallas_call(
        _decode_body,
        out_shape=[jax.ShapeDtypeStruct((NUM_CLASSES, _NP), jnp.float32),
                   jax.ShapeDtypeStruct((5, _NP), jnp.float32)],
    )(t, consts)

    mesh = plsc.VectorSubcoreMesh(core_axis_name="c", subcore_axis_name="s",
                                  num_cores=2, num_subcores=16)
    out = pl.kernel(
        _sc_nms_body,
        out_type=jax.ShapeDtypeStruct((NUM_CLASSES * 160,), jnp.float32),
        mesh=mesh,
        compiler_params=pltpu.CompilerParams(needs_layout_passes=False),
        scratch_types=[
            pltpu.VMEM((_NP,), jnp.float32),
            pltpu.VMEM((_NP,), jnp.float32),
            pltpu.VMEM((_NP,), jnp.float32),
            pltpu.VMEM((_NP,), jnp.float32),
            pltpu.VMEM((_NP,), jnp.float32),
            pltpu.VMEM((_NP,), jnp.float32),
            pltpu.VMEM((_NPC,), jnp.float32),
            pltpu.VMEM((_NPC,), jnp.int32),
            pltpu.VMEM((192,), jnp.float32),
            pltpu.VMEM((16,), jnp.float32),
            pltpu.VMEM((16,), jnp.int32),
        ],
    )(s_mat.reshape(-1), b_mat.reshape(-1))

    sel = out.reshape(NUM_CLASSES, 5, 32)[:, :, :MAX_BOXES]
    boxes_ = jnp.stack([sel[:, 1], sel[:, 2], sel[:, 3], sel[:, 4]],
                       axis=-1).reshape(-1, 4)
    scores_ = sel[:, 0].reshape(-1)
    classes_ = jnp.repeat(jnp.arange(NUM_CLASSES, dtype=jnp.int32), MAX_BOXES)
    return boxes_, scores_, classes_


# speedup vs baseline: 1.4160x; 1.2220x over previous
import numpy as np

import jax
import jax.numpy as jnp
from jax import lax
from jax.experimental import pallas as pl
from jax.experimental.pallas import tpu as pltpu
from jax.experimental.pallas import tpu_sc as plsc

NUM_CLASSES = 80
MAX_BOXES = 20
SCORE_THR = 0.6
IOU_THR = 0.5

_G0, _G1 = 19, 38
_N0 = _G0 * _G0 * 3
_N1 = _G1 * _G1 * 3
_N = _N0 + _N1
_NP = 43 * 128
_NCHUNK = _NP // 16
_NPC = _NP + 32

_ANCHORS = np.array([[10.0, 14.0], [23.0, 27.0], [37.0, 58.0],
                     [81.0, 82.0], [135.0, 169.0], [344.0, 319.0]],
                    dtype=np.float32)
_MASK0 = [3, 4, 5]
_MASK1 = [1, 2, 3]

_INPUT = np.float32(_G0 * 32)
_IMG = np.array([720.0, 1280.0], dtype=np.float32)
_SC_ = np.float32(min(np.float32(_INPUT / _IMG[0]), np.float32(_INPUT / _IMG[1])))
_NEW = np.round(_IMG * _SC_).astype(np.float32)
_OFF = ((_INPUT - _NEW) / np.float32(2.0) / _INPUT).astype(np.float32)
_SCALE = (_INPUT / _NEW).astype(np.float32)


def _make_consts():
    gx = np.zeros(_NP, np.float32)
    gy = np.zeros(_NP, np.float32)
    gd = np.ones(_NP, np.float32)
    aw = np.zeros(_NP, np.float32)
    ah = np.zeros(_NP, np.float32)
    n = np.arange(_N0)
    cell, a = n // 3, n % 3
    gx[:_N0] = (cell % _G0).astype(np.float32)
    gy[:_N0] = (cell // _G0).astype(np.float32)
    gd[:_N0] = float(_G0)
    anc = _ANCHORS[_MASK0][a]
    aw[:_N0] = anc[:, 0]
    ah[:_N0] = anc[:, 1]
    n = np.arange(_N1)
    cell, a = n // 3, n % 3
    gx[_N0:_N] = (cell % _G1).astype(np.float32)
    gy[_N0:_N] = (cell // _G1).astype(np.float32)
    gd[_N0:_N] = float(_G1)
    anc = _ANCHORS[_MASK1][a]
    aw[_N0:_N] = anc[:, 0]
    ah[_N0:_N] = anc[:, 1]
    return np.stack([gx, gy, gd, aw, ah], axis=0)


_CONSTS = _make_consts()


def _decode_body(t_ref, c_ref, s_out, b_out):
    def sig(x):
        return 1.0 / (1.0 + jnp.exp(-x))

    tx = t_ref[0:1, :]
    ty = t_ref[1:2, :]
    tw = t_ref[2:3, :]
    th = t_ref[3:4, :]
    tc = t_ref[4:5, :]
    tp = t_ref[5:85, :]
    gx = c_ref[0:1, :]
    gy = c_ref[1:2, :]
    gd = c_ref[2:3, :]
    aw = c_ref[3:4, :]
    ah = c_ref[4:5, :]

    bx = (sig(tx) + gx) / gd
    by = (sig(ty) + gy) / gd
    bw = jnp.exp(tw) * aw / _INPUT
    bh = jnp.exp(th) * ah / _INPUT
    yy = (by - _OFF[0]) * _SCALE[0]
    xx = (bx - _OFF[1]) * _SCALE[1]
    hh = bh * _SCALE[0]
    ww = bw * _SCALE[1]
    y1 = (yy - hh / 2.0) * _IMG[0]
    x1 = (xx - ww / 2.0) * _IMG[1]
    y2 = (yy + hh / 2.0) * _IMG[0]
    x2 = (xx + ww / 2.0) * _IMG[1]
    area = (y2 - y1) * (x2 - x1)

    s = sig(tc) * sig(tp)
    s = jnp.where(s >= SCORE_THR, s, 0.0)

    s_out[:, :] = s
    b_out[0:1, :] = y1
    b_out[1:2, :] = x1
    b_out[2:3, :] = y2
    b_out[3:4, :] = x2
    b_out[4:5, :] = area


def _sc_nms_body(s_hbm, b_hbm, out_hbm,
                 s_v, y1_v, x1_v, y2_v, x2_v, ar_v, cs_v, ci_v, sel_v,
                 shf_v, shi_v):
    wid = lax.axis_index("s") * 2 + lax.axis_index("c")
    lane = jnp.arange(16, dtype=jnp.int32)

    def bmax(x):
        for s in (8, 4, 2, 1):
            shf_v[pl.ds(0, 16)] = x
            x = jnp.maximum(x, plsc.load_gather(shf_v, [lane ^ s]))
        return x

    def bmin(x):
        for s in (8, 4, 2, 1):
            shi_v[pl.ds(0, 16)] = x
            x = jnp.minimum(x, plsc.load_gather(shi_v, [lane ^ s]))
        return x
    zero16 = jnp.zeros((16,), jnp.float32)
    izero16 = jnp.zeros((16,), jnp.int32)
    bv0 = jnp.full((16,), -1.0, jnp.float32)
    bi0 = jnp.zeros((16,), jnp.int32)

    pltpu.sync_copy(b_hbm.at[pl.ds(0 * _NP, _NP)], y1_v)
    pltpu.sync_copy(b_hbm.at[pl.ds(1 * _NP, _NP)], x1_v)
    pltpu.sync_copy(b_hbm.at[pl.ds(2 * _NP, _NP)], y2_v)
    pltpu.sync_copy(b_hbm.at[pl.ds(3 * _NP, _NP)], x2_v)
    pltpu.sync_copy(b_hbm.at[pl.ds(4 * _NP, _NP)], ar_v)

    def do_class(c):
        pltpu.sync_copy(s_hbm.at[pl.ds(c * _NP, _NP)], s_v)
        for k in range(10):
            sel_v[pl.ds(k * 16, 16)] = zero16

        @pl.loop(0, _NCHUNK, init_carry=jnp.int32(0), unroll=4)
        def comp(k, cnt):
            v = s_v[pl.ds(k * 16, 16)]
            msk = v > 0.0
            plsc.store_compressed(cs_v.at[pl.ds(cnt, 16)], v, mask=msk)
            plsc.store_compressed(ci_v.at[pl.ds(cnt, 16)], k * 16 + lane,
                                  mask=msk)
            n = plsc.all_reduce_population_count(msk)
            return cnt + n[0]

        cnt = comp
        plsc.store_scatter(cs_v, [cnt + lane], zero16)
        plsc.store_scatter(cs_v, [cnt + 16 + lane], zero16)
        plsc.store_scatter(ci_v, [cnt + lane], izero16)
        plsc.store_scatter(ci_v, [cnt + 16 + lane], izero16)
        nb = (cnt + 15) // 16

        @pl.loop(0, nb, init_carry=(bv0, bi0), unroll=1)
        def amx(k, carry):
            bv, bi = carry
            v = cs_v[pl.ds(k * 16, 16)]
            gt = v > bv
            return jnp.where(gt, v, bv), jnp.where(gt, k, bi)

        bv, bi = amx

        @pl.loop(0, MAX_BOXES, init_carry=(bv, bi), unroll=1)
        def rnd(i, carry):
            bv, bi = carry
            mx_v = bmax(bv)
            idxv = bi * 16 + lane
            cand_v = bmin(jnp.where(bv == mx_v, idxv, jnp.int32(1 << 30)))
            keep_v = mx_v > 0.0
            oi_v = plsc.load_gather(ci_v, [cand_v])
            ky1 = plsc.load_gather(y1_v, [oi_v])
            kx1 = plsc.load_gather(x1_v, [oi_v])
            ky2 = plsc.load_gather(y2_v, [oi_v])
            kx2 = plsc.load_gather(x2_v, [oi_v])
            kar = plsc.load_gather(ar_v, [oi_v])
            val = jnp.where(lane == 0, mx_v,
                            jnp.where(lane == 1, ky1,
                                      jnp.where(lane == 2, kx1,
                                                jnp.where(lane == 3, ky2, kx2))))
            selpos = jnp.where((lane < 5) & keep_v, lane * 32 + i,
                               jnp.int32(176) + lane)
            plsc.store_scatter(sel_v, [selpos], val)

            @pl.loop(0, nb, init_carry=(bv0, bi0), unroll=1)
            def sup(k, carry2):
                nv, ni = carry2
                sl = pl.ds(k * 16, 16)
                sv = cs_v[sl]
                gi = ci_v[sl]
                vy1 = plsc.load_gather(y1_v, [gi])
                vx1 = plsc.load_gather(x1_v, [gi])
                vy2 = plsc.load_gather(y2_v, [gi])
                vx2 = plsc.load_gather(x2_v, [gi])
                var_ = plsc.load_gather(ar_v, [gi])
                iy1 = jnp.maximum(ky1, vy1)
                ix1 = jnp.maximum(kx1, vx1)
                iy2 = jnp.minimum(ky2, vy2)
                ix2 = jnp.minimum(kx2, vx2)
                inter = jnp.maximum(iy2 - iy1, 0.0) * jnp.maximum(ix2 - ix1, 0.0)
                iou = inter / (kar + var_ - inter + 1e-9)
                pos = k * 16 + lane
                supm = (iou > IOU_THR) | (pos == cand_v)
                s_upd = jnp.where(supm & keep_v, 0.0, sv)
                cs_v[sl] = s_upd
                gt = s_upd > nv
                return jnp.where(gt, s_upd, nv), jnp.where(gt, k, ni)

            return sup

        pltpu.sync_copy(sel_v.at[pl.ds(0, 160)], out_hbm.at[pl.ds(c * 160, 160)])

    @pl.loop(0, 3, unroll=1)
    def rep_body(rep):
        c = wid + rep * 32

        @pl.when(c < NUM_CLASSES)
        def _():
            do_class(c)


def kernel(feat0, feat1, image_shape_t):
    del image_shape_t
    f0 = feat0.reshape(_N0, NUM_CLASSES + 5)
    f1 = feat1.reshape(_N1, NUM_CLASSES + 5)
    t = jnp.concatenate([f0, f1], axis=0)
    t = jnp.pad(t, ((0, _NP - _N), (0, 0)), constant_values=-1e9).T
    consts = jnp.asarray(_CONSTS)

    s_mat, b_mat = pl.pallas_call(
        _decode_body,
        out_shape=[jax.ShapeDtypeStruct((NUM_CLASSES, _NP), jnp.float32),
                   jax.ShapeDtypeStruct((5, _NP), jnp.float32)],
    )(t, consts)

    mesh = plsc.VectorSubcoreMesh(core_axis_name="c", subcore_axis_name="s",
                                  num_cores=2, num_subcores=16)
    out = pl.kernel(
        _sc_nms_body,
        out_type=jax.ShapeDtypeStruct((NUM_CLASSES * 160,), jnp.float32),
        mesh=mesh,
        compiler_params=pltpu.CompilerParams(needs_layout_passes=False),
        scratch_types=[
            pltpu.VMEM((_NP,), jnp.float32),
            pltpu.VMEM((_NP,), jnp.float32),
            pltpu.VMEM((_NP,), jnp.float32),
            pltpu.VMEM((_NP,), jnp.float32),
            pltpu.VMEM((_NP,), jnp.float32),
            pltpu.VMEM((_NP,), jnp.float32),
            pltpu.VMEM((_NPC,), jnp.float32),
            pltpu.VMEM((_NPC,), jnp.int32),
            pltpu.VMEM((192,), jnp.float32),
            pltpu.VMEM((16,), jnp.float32),
            pltpu.VMEM((16,), jnp.int32),
        ],
    )(s_mat.reshape(-1), b_mat.reshape(-1))

    sel = out.reshape(NUM_CLASSES, 5, 32)[:, :, :MAX_BOXES]
    boxes_ = jnp.stack([sel[:, 1], sel[:, 2], sel[:, 3], sel[:, 4]],
                       axis=-1).reshape(-1, 4)
    scores_ = sel[:, 0].reshape(-1)
    classes_ = jnp.repeat(jnp.arange(NUM_CLASSES, dtype=jnp.int32), MAX_BOXES)
    return boxes_, scores_, classes_


# speedup vs baseline: 1.4394x; 1.0166x over previous
import numpy as np

import jax
import jax.numpy as jnp
from jax import lax
from jax.experimental import pallas as pl
from jax.experimental.pallas import tpu as pltpu
from jax.experimental.pallas import tpu_sc as plsc

NUM_CLASSES = 80
MAX_BOXES = 20
SCORE_THR = 0.6
IOU_THR = 0.5

_G0, _G1 = 19, 38
_N0 = _G0 * _G0 * 3
_N1 = _G1 * _G1 * 3
_N = _N0 + _N1
_NP = 43 * 128
_NCHUNK = _NP // 16
_NPC = _NP + 32

_ANCHORS = np.array([[10.0, 14.0], [23.0, 27.0], [37.0, 58.0],
                     [81.0, 82.0], [135.0, 169.0], [344.0, 319.0]],
                    dtype=np.float32)
_MASK0 = [3, 4, 5]
_MASK1 = [1, 2, 3]

_INPUT = np.float32(_G0 * 32)
_IMG = np.array([720.0, 1280.0], dtype=np.float32)
_SC_ = np.float32(min(np.float32(_INPUT / _IMG[0]), np.float32(_INPUT / _IMG[1])))
_NEW = np.round(_IMG * _SC_).astype(np.float32)
_OFF = ((_INPUT - _NEW) / np.float32(2.0) / _INPUT).astype(np.float32)
_SCALE = (_INPUT / _NEW).astype(np.float32)


def _make_consts():
    gx = np.zeros(_NP, np.float32)
    gy = np.zeros(_NP, np.float32)
    gd = np.ones(_NP, np.float32)
    aw = np.zeros(_NP, np.float32)
    ah = np.zeros(_NP, np.float32)
    n = np.arange(_N0)
    cell, a = n // 3, n % 3
    gx[:_N0] = (cell % _G0).astype(np.float32)
    gy[:_N0] = (cell // _G0).astype(np.float32)
    gd[:_N0] = float(_G0)
    anc = _ANCHORS[_MASK0][a]
    aw[:_N0] = anc[:, 0]
    ah[:_N0] = anc[:, 1]
    n = np.arange(_N1)
    cell, a = n // 3, n % 3
    gx[_N0:_N] = (cell % _G1).astype(np.float32)
    gy[_N0:_N] = (cell // _G1).astype(np.float32)
    gd[_N0:_N] = float(_G1)
    anc = _ANCHORS[_MASK1][a]
    aw[_N0:_N] = anc[:, 0]
    ah[_N0:_N] = anc[:, 1]
    return np.stack([gx, gy, gd, aw, ah], axis=0)


_CONSTS = _make_consts()


def _decode_body(t_ref, c_ref, s_out, b_out):
    def sig(x):
        return 1.0 / (1.0 + jnp.exp(-x))

    tx = t_ref[0:1, :]
    ty = t_ref[1:2, :]
    tw = t_ref[2:3, :]
    th = t_ref[3:4, :]
    tc = t_ref[4:5, :]
    tp = t_ref[5:85, :]
    gx = c_ref[0:1, :]
    gy = c_ref[1:2, :]
    gd = c_ref[2:3, :]
    aw = c_ref[3:4, :]
    ah = c_ref[4:5, :]

    bx = (sig(tx) + gx) / gd
    by = (sig(ty) + gy) / gd
    bw = jnp.exp(tw) * aw / _INPUT
    bh = jnp.exp(th) * ah / _INPUT
    yy = (by - _OFF[0]) * _SCALE[0]
    xx = (bx - _OFF[1]) * _SCALE[1]
    hh = bh * _SCALE[0]
    ww = bw * _SCALE[1]
    y1 = (yy - hh / 2.0) * _IMG[0]
    x1 = (xx - ww / 2.0) * _IMG[1]
    y2 = (yy + hh / 2.0) * _IMG[0]
    x2 = (xx + ww / 2.0) * _IMG[1]
    area = (y2 - y1) * (x2 - x1)

    s = sig(tc) * sig(tp)
    s = jnp.where(s >= SCORE_THR, s, 0.0)

    s_out[:, :] = s
    b_out[0:1, :] = y1
    b_out[1:2, :] = x1
    b_out[2:3, :] = y2
    b_out[3:4, :] = x2
    b_out[4:5, :] = area


def _sc_nms_body(s_hbm, b_hbm, out_hbm,
                 s_v, y1_v, x1_v, y2_v, x2_v, ar_v, cs_v, ci_v, sel_v,
                 shf_v, shi_v):
    wid = lax.axis_index("s") * 2 + lax.axis_index("c")
    lane = jnp.arange(16, dtype=jnp.int32)

    lane15 = jnp.full((16,), 15, jnp.int32)

    def bmax(x):
        shf_v[pl.ds(0, 16)] = plsc.cummax(x)
        return plsc.load_gather(shf_v, [lane15])

    def bmin(x):
        shi_v[pl.ds(0, 16)] = plsc.cummax(-x)
        return -plsc.load_gather(shi_v, [lane15])
    zero16 = jnp.zeros((16,), jnp.float32)
    izero16 = jnp.zeros((16,), jnp.int32)
    bv0 = jnp.full((16,), -1.0, jnp.float32)
    bi0 = jnp.zeros((16,), jnp.int32)

    pltpu.sync_copy(b_hbm.at[pl.ds(0 * _NP, _NP)], y1_v)
    pltpu.sync_copy(b_hbm.at[pl.ds(1 * _NP, _NP)], x1_v)
    pltpu.sync_copy(b_hbm.at[pl.ds(2 * _NP, _NP)], y2_v)
    pltpu.sync_copy(b_hbm.at[pl.ds(3 * _NP, _NP)], x2_v)
    pltpu.sync_copy(b_hbm.at[pl.ds(4 * _NP, _NP)], ar_v)

    def do_class(c):
        pltpu.sync_copy(s_hbm.at[pl.ds(c * _NP, _NP)], s_v)
        for k in range(10):
            sel_v[pl.ds(k * 16, 16)] = zero16

        @pl.loop(0, _NCHUNK, init_carry=jnp.int32(0), unroll=4)
        def comp(k, cnt):
            v = s_v[pl.ds(k * 16, 16)]
            msk = v > 0.0
            plsc.store_compressed(cs_v.at[pl.ds(cnt, 16)], v, mask=msk)
            plsc.store_compressed(ci_v.at[pl.ds(cnt, 16)], k * 16 + lane,
                                  mask=msk)
            n = plsc.all_reduce_population_count(msk)
            return cnt + n[0]

        cnt = comp
        plsc.store_scatter(cs_v, [cnt + lane], zero16)
        plsc.store_scatter(cs_v, [cnt + 16 + lane], zero16)
        plsc.store_scatter(ci_v, [cnt + lane], izero16)
        plsc.store_scatter(ci_v, [cnt + 16 + lane], izero16)
        nb = (cnt + 15) // 16

        @pl.loop(0, nb, init_carry=(bv0, bi0), unroll=1)
        def amx(k, carry):
            bv, bi = carry
            v = cs_v[pl.ds(k * 16, 16)]
            gt = v > bv
            return jnp.where(gt, v, bv), jnp.where(gt, k, bi)

        bv, bi = amx

        @pl.loop(0, MAX_BOXES, init_carry=(bv, bi), unroll=1)
        def rnd(i, carry):
            bv, bi = carry
            mx_v = bmax(bv)
            idxv = bi * 16 + lane
            cand_v = bmin(jnp.where(bv == mx_v, idxv, jnp.int32(1 << 30)))
            keep_v = mx_v > 0.0
            oi_v = plsc.load_gather(ci_v, [cand_v])
            ky1 = plsc.load_gather(y1_v, [oi_v])
            kx1 = plsc.load_gather(x1_v, [oi_v])
            ky2 = plsc.load_gather(y2_v, [oi_v])
            kx2 = plsc.load_gather(x2_v, [oi_v])
            kar = plsc.load_gather(ar_v, [oi_v])
            val = jnp.where(lane == 0, mx_v,
                            jnp.where(lane == 1, ky1,
                                      jnp.where(lane == 2, kx1,
                                                jnp.where(lane == 3, ky2, kx2))))
            selpos = jnp.where((lane < 5) & keep_v, lane * 32 + i,
                               jnp.int32(176) + lane)
            plsc.store_scatter(sel_v, [selpos], val)

            @pl.loop(0, nb, init_carry=(bv0, bi0), unroll=1)
            def sup(k, carry2):
                nv, ni = carry2
                sl = pl.ds(k * 16, 16)
                sv = cs_v[sl]
                gi = ci_v[sl]
                vy1 = plsc.load_gather(y1_v, [gi])
                vx1 = plsc.load_gather(x1_v, [gi])
                vy2 = plsc.load_gather(y2_v, [gi])
                vx2 = plsc.load_gather(x2_v, [gi])
                var_ = plsc.load_gather(ar_v, [gi])
                iy1 = jnp.maximum(ky1, vy1)
                ix1 = jnp.maximum(kx1, vx1)
                iy2 = jnp.minimum(ky2, vy2)
                ix2 = jnp.minimum(kx2, vx2)
                inter = jnp.maximum(iy2 - iy1, 0.0) * jnp.maximum(ix2 - ix1, 0.0)
                iou = inter / (kar + var_ - inter + 1e-9)
                pos = k * 16 + lane
                supm = (iou > IOU_THR) | (pos == cand_v)
                s_upd = jnp.where(supm & keep_v, 0.0, sv)
                cs_v[sl] = s_upd
                gt = s_upd > nv
                return jnp.where(gt, s_upd, nv), jnp.where(gt, k, ni)

            return sup

        pltpu.sync_copy(sel_v.at[pl.ds(0, 160)], out_hbm.at[pl.ds(c * 160, 160)])

    @pl.loop(0, 3, unroll=1)
    def rep_body(rep):
        c = wid + rep * 32

        @pl.when(c < NUM_CLASSES)
        def _():
            do_class(c)


def kernel(feat0, feat1, image_shape_t):
    del image_shape_t
    f0 = feat0.reshape(_N0, NUM_CLASSES + 5)
    f1 = feat1.reshape(_N1, NUM_CLASSES + 5)
    t = jnp.concatenate([f0, f1], axis=0)
    t = jnp.pad(t, ((0, _NP - _N), (0, 0)), constant_values=-1e9).T
    consts = jnp.asarray(_CONSTS)

    s_mat, b_mat = pl.pallas_call(
        _decode_body,
        out_shape=[jax.ShapeDtypeStruct((NUM_CLASSES, _NP), jnp.float32),
                   jax.ShapeDtypeStruct((5, _NP), jnp.float32)],
    )(t, consts)

    mesh = plsc.VectorSubcoreMesh(core_axis_name="c", subcore_axis_name="s",
                                  num_cores=2, num_subcores=16)
    out = pl.kernel(
        _sc_nms_body,
        out_type=jax.ShapeDtypeStruct((NUM_CLASSES * 160,), jnp.float32),
        mesh=mesh,
        compiler_params=pltpu.CompilerParams(needs_layout_passes=False),
        scratch_types=[
            pltpu.VMEM((_NP,), jnp.float32),
            pltpu.VMEM((_NP,), jnp.float32),
            pltpu.VMEM((_NP,), jnp.float32),
            pltpu.VMEM((_NP,), jnp.float32),
            pltpu.VMEM((_NP,), jnp.float32),
            pltpu.VMEM((_NP,), jnp.float32),
            pltpu.VMEM((_NPC,), jnp.float32),
            pltpu.VMEM((_NPC,), jnp.int32),
            pltpu.VMEM((192,), jnp.float32),
            pltpu.VMEM((16,), jnp.float32),
            pltpu.VMEM((16,), jnp.int32),
        ],
    )(s_mat.reshape(-1), b_mat.reshape(-1))

    sel = out.reshape(NUM_CLASSES, 5, 32)[:, :, :MAX_BOXES]
    boxes_ = jnp.stack([sel[:, 1], sel[:, 2], sel[:, 3], sel[:, 4]],
                       axis=-1).reshape(-1, 4)
    scores_ = sel[:, 0].reshape(-1)
    classes_ = jnp.repeat(jnp.arange(NUM_CLASSES, dtype=jnp.int32), MAX_BOXES)
    return boxes_, scores_, classes_


# speedup vs baseline: 1.5291x; 1.0623x over previous
import numpy as np

import jax
import jax.numpy as jnp
from jax import lax
from jax.experimental import pallas as pl
from jax.experimental.pallas import tpu as pltpu
from jax.experimental.pallas import tpu_sc as plsc

NUM_CLASSES = 80
MAX_BOXES = 20
SCORE_THR = 0.6
IOU_THR = 0.5

_G0, _G1 = 19, 38
_N0 = _G0 * _G0 * 3
_N1 = _G1 * _G1 * 3
_N = _N0 + _N1
_NP = 43 * 128
_NCHUNK = _NP // 16
_NPC = _NP + 32

_ANCHORS = np.array([[10.0, 14.0], [23.0, 27.0], [37.0, 58.0],
                     [81.0, 82.0], [135.0, 169.0], [344.0, 319.0]],
                    dtype=np.float32)
_MASK0 = [3, 4, 5]
_MASK1 = [1, 2, 3]

_INPUT = np.float32(_G0 * 32)
_IMG = np.array([720.0, 1280.0], dtype=np.float32)
_SC_ = np.float32(min(np.float32(_INPUT / _IMG[0]), np.float32(_INPUT / _IMG[1])))
_NEW = np.round(_IMG * _SC_).astype(np.float32)
_OFF = ((_INPUT - _NEW) / np.float32(2.0) / _INPUT).astype(np.float32)
_SCALE = (_INPUT / _NEW).astype(np.float32)


def _make_consts():
    gx = np.zeros(_NP, np.float32)
    gy = np.zeros(_NP, np.float32)
    gd = np.ones(_NP, np.float32)
    aw = np.zeros(_NP, np.float32)
    ah = np.zeros(_NP, np.float32)
    n = np.arange(_N0)
    cell, a = n // 3, n % 3
    gx[:_N0] = (cell % _G0).astype(np.float32)
    gy[:_N0] = (cell // _G0).astype(np.float32)
    gd[:_N0] = float(_G0)
    anc = _ANCHORS[_MASK0][a]
    aw[:_N0] = anc[:, 0]
    ah[:_N0] = anc[:, 1]
    n = np.arange(_N1)
    cell, a = n // 3, n % 3
    gx[_N0:_N] = (cell % _G1).astype(np.float32)
    gy[_N0:_N] = (cell // _G1).astype(np.float32)
    gd[_N0:_N] = float(_G1)
    anc = _ANCHORS[_MASK1][a]
    aw[_N0:_N] = anc[:, 0]
    ah[_N0:_N] = anc[:, 1]
    return np.stack([gx, gy, gd, aw, ah], axis=0)


_CONSTS = _make_consts()


def _decode_body(t_ref, c_ref, s_out, b_out):
    def sig(x):
        return 1.0 / (1.0 + jnp.exp(-x))

    tx = t_ref[0:1, :]
    ty = t_ref[1:2, :]
    tw = t_ref[2:3, :]
    th = t_ref[3:4, :]
    tc = t_ref[4:5, :]
    tp = t_ref[5:85, :]
    gx = c_ref[0:1, :]
    gy = c_ref[1:2, :]
    gd = c_ref[2:3, :]
    aw = c_ref[3:4, :]
    ah = c_ref[4:5, :]

    bx = (sig(tx) + gx) / gd
    by = (sig(ty) + gy) / gd
    bw = jnp.exp(tw) * aw / _INPUT
    bh = jnp.exp(th) * ah / _INPUT
    yy = (by - _OFF[0]) * _SCALE[0]
    xx = (bx - _OFF[1]) * _SCALE[1]
    hh = bh * _SCALE[0]
    ww = bw * _SCALE[1]
    y1 = (yy - hh / 2.0) * _IMG[0]
    x1 = (xx - ww / 2.0) * _IMG[1]
    y2 = (yy + hh / 2.0) * _IMG[0]
    x2 = (xx + ww / 2.0) * _IMG[1]
    area = (y2 - y1) * (x2 - x1)

    s = sig(tc) * sig(tp)
    s = jnp.where(s >= SCORE_THR, s, 0.0)

    s_out[:, :] = s
    b_out[0:1, :] = y1
    b_out[1:2, :] = x1
    b_out[2:3, :] = y2
    b_out[3:4, :] = x2
    b_out[4:5, :] = area


def _sc_nms_body(s_hbm, b_hbm, out_hbm,
                 s_v, y1_v, x1_v, y2_v, x2_v, ar_v, cs_v, ci_v, sel_v,
                 shf_v, shi_v, cy1_v, cx1_v, cy2_v, cx2_v, car_v):
    wid = lax.axis_index("s") * 2 + lax.axis_index("c")
    lane = jnp.arange(16, dtype=jnp.int32)

    lane15 = jnp.full((16,), 15, jnp.int32)

    def bmax(x):
        shf_v[pl.ds(0, 16)] = plsc.cummax(x)
        return plsc.load_gather(shf_v, [lane15])

    def bmin(x):
        shi_v[pl.ds(0, 16)] = plsc.cummax(-x)
        return -plsc.load_gather(shi_v, [lane15])
    zero16 = jnp.zeros((16,), jnp.float32)
    izero16 = jnp.zeros((16,), jnp.int32)
    bv0 = jnp.full((16,), -1.0, jnp.float32)
    bi0 = jnp.zeros((16,), jnp.int32)

    pltpu.sync_copy(b_hbm.at[pl.ds(0 * _NP, _NP)], y1_v)
    pltpu.sync_copy(b_hbm.at[pl.ds(1 * _NP, _NP)], x1_v)
    pltpu.sync_copy(b_hbm.at[pl.ds(2 * _NP, _NP)], y2_v)
    pltpu.sync_copy(b_hbm.at[pl.ds(3 * _NP, _NP)], x2_v)
    pltpu.sync_copy(b_hbm.at[pl.ds(4 * _NP, _NP)], ar_v)

    def do_class(c):
        pltpu.sync_copy(s_hbm.at[pl.ds(c * _NP, _NP)], s_v)
        for k in range(10):
            sel_v[pl.ds(k * 16, 16)] = zero16

        @pl.loop(0, _NCHUNK, init_carry=jnp.int32(0), unroll=4)
        def comp(k, cnt):
            v = s_v[pl.ds(k * 16, 16)]
            msk = v > 0.0
            plsc.store_compressed(cs_v.at[pl.ds(cnt, 16)], v, mask=msk)
            plsc.store_compressed(ci_v.at[pl.ds(cnt, 16)], k * 16 + lane,
                                  mask=msk)
            n = plsc.all_reduce_population_count(msk)
            return cnt + n[0]

        cnt = comp
        plsc.store_scatter(cs_v, [cnt + lane], zero16)
        plsc.store_scatter(cs_v, [cnt + 16 + lane], zero16)
        plsc.store_scatter(ci_v, [cnt + lane], izero16)
        plsc.store_scatter(ci_v, [cnt + 16 + lane], izero16)
        nb = (cnt + 15) // 16

        @pl.loop(0, nb, init_carry=(bv0, bi0), unroll=1)
        def amx(k, carry):
            bv, bi = carry
            sl = pl.ds(k * 16, 16)
            gi = ci_v[sl]
            cy1_v[sl] = plsc.load_gather(y1_v, [gi])
            cx1_v[sl] = plsc.load_gather(x1_v, [gi])
            cy2_v[sl] = plsc.load_gather(y2_v, [gi])
            cx2_v[sl] = plsc.load_gather(x2_v, [gi])
            car_v[sl] = plsc.load_gather(ar_v, [gi])
            v = cs_v[sl]
            gt = v > bv
            return jnp.where(gt, v, bv), jnp.where(gt, k, bi)

        bv, bi = amx

        @pl.loop(0, MAX_BOXES, init_carry=(bv, bi), unroll=1)
        def rnd(i, carry):
            bv, bi = carry
            mx_v = bmax(bv)
            idxv = bi * 16 + lane
            cand_v = bmin(jnp.where(bv == mx_v, idxv, jnp.int32(1 << 30)))
            keep_v = mx_v > 0.0
            ky1 = plsc.load_gather(cy1_v, [cand_v])
            kx1 = plsc.load_gather(cx1_v, [cand_v])
            ky2 = plsc.load_gather(cy2_v, [cand_v])
            kx2 = plsc.load_gather(cx2_v, [cand_v])
            kar = plsc.load_gather(car_v, [cand_v])
            val = jnp.where(lane == 0, mx_v,
                            jnp.where(lane == 1, ky1,
                                      jnp.where(lane == 2, kx1,
                                                jnp.where(lane == 3, ky2, kx2))))
            selpos = jnp.where((lane < 5) & keep_v, lane * 32 + i,
                               jnp.int32(176) + lane)
            plsc.store_scatter(sel_v, [selpos], val)

            @pl.loop(0, nb, init_carry=(bv0, bi0), unroll=1)
            def sup(k, carry2):
                nv, ni = carry2
                sl = pl.ds(k * 16, 16)
                sv = cs_v[sl]
                vy1 = cy1_v[sl]
                vx1 = cx1_v[sl]
                vy2 = cy2_v[sl]
                vx2 = cx2_v[sl]
                var_ = car_v[sl]
                iy1 = jnp.maximum(ky1, vy1)
                ix1 = jnp.maximum(kx1, vx1)
                iy2 = jnp.minimum(ky2, vy2)
                ix2 = jnp.minimum(kx2, vx2)
                inter = jnp.maximum(iy2 - iy1, 0.0) * jnp.maximum(ix2 - ix1, 0.0)
                iou = inter / (kar + var_ - inter + 1e-9)
                pos = k * 16 + lane
                supm = (iou > IOU_THR) | (pos == cand_v)
                s_upd = jnp.where(supm & keep_v, 0.0, sv)
                cs_v[sl] = s_upd
                gt = s_upd > nv
                return jnp.where(gt, s_upd, nv), jnp.where(gt, k, ni)

            return sup

        pltpu.sync_copy(sel_v.at[pl.ds(0, 160)], out_hbm.at[pl.ds(c * 160, 160)])

    @pl.loop(0, 3, unroll=1)
    def rep_body(rep):
        c = wid + rep * 32

        @pl.when(c < NUM_CLASSES)
        def _():
            do_class(c)


def kernel(feat0, feat1, image_shape_t):
    del image_shape_t
    f0 = feat0.reshape(_N0, NUM_CLASSES + 5)
    f1 = feat1.reshape(_N1, NUM_CLASSES + 5)
    t = jnp.concatenate([f0, f1], axis=0)
    t = jnp.pad(t, ((0, _NP - _N), (0, 0)), constant_values=-1e9).T
    consts = jnp.asarray(_CONSTS)

    s_mat, b_mat = pl.pallas_call(
        _decode_body,
        out_shape=[jax.ShapeDtypeStruct((NUM_CLASSES, _NP), jnp.float32),
                   jax.ShapeDtypeStruct((5, _NP), jnp.float32)],
    )(t, consts)

    mesh = plsc.VectorSubcoreMesh(core_axis_name="c", subcore_axis_name="s",
                                  num_cores=2, num_subcores=16)
    out = pl.kernel(
        _sc_nms_body,
        out_type=jax.ShapeDtypeStruct((NUM_CLASSES * 160,), jnp.float32),
        mesh=mesh,
        compiler_params=pltpu.CompilerParams(needs_layout_passes=False),
        scratch_types=[
            pltpu.VMEM((_NP,), jnp.float32),
            pltpu.VMEM((_NP,), jnp.float32),
            pltpu.VMEM((_NP,), jnp.float32),
            pltpu.VMEM((_NP,), jnp.float32),
            pltpu.VMEM((_NP,), jnp.float32),
            pltpu.VMEM((_NP,), jnp.float32),
            pltpu.VMEM((_NPC,), jnp.float32),
            pltpu.VMEM((_NPC,), jnp.int32),
            pltpu.VMEM((192,), jnp.float32),
            pltpu.VMEM((16,), jnp.float32),
            pltpu.VMEM((16,), jnp.int32),
            pltpu.VMEM((_NPC,), jnp.float32),
            pltpu.VMEM((_NPC,), jnp.float32),
            pltpu.VMEM((_NPC,), jnp.float32),
            pltpu.VMEM((_NPC,), jnp.float32),
            pltpu.VMEM((_NPC,), jnp.float32),
        ],
    )(s_mat.reshape(-1), b_mat.reshape(-1))

    sel = out.reshape(NUM_CLASSES, 5, 32)[:, :, :MAX_BOXES]
    boxes_ = jnp.stack([sel[:, 1], sel[:, 2], sel[:, 3], sel[:, 4]],
                       axis=-1).reshape(-1, 4)
    scores_ = sel[:, 0].reshape(-1)
    classes_ = jnp.repeat(jnp.arange(NUM_CLASSES, dtype=jnp.int32), MAX_BOXES)
    return boxes_, scores_, classes_
